# Initial kernel scaffold; baseline (speedup 1.0000x reference)
import jax
import jax.numpy as jnp
from jax.experimental import pallas as pl


def kernel(x_0, n0_to_0, cci_0_to_0, global_feature, batch_0, W1, W2, fc1_w, fc1_b, fc2_w, fc2_b, fc3_w, fc3_b, fc4_w, fc4_b):
    raise NotImplementedError("write your pallas kernel here")


# speedup vs baseline: 4.0273x; 4.0273x over previous
import dataclasses
import functools

import jax
import jax.numpy as jnp
from jax import lax
from jax.experimental import pallas as pl
from jax.experimental.pallas import tpu as pltpu
from jax.experimental.pallas import tpu_sc as plsc

N = 10000
NPAD = 10240
E = 320000
D = 128
G = 64

NC = 2
NS = 16
NT = NC * NS
RPT = 79
EPT = RPT * 128
EPAD = NT * EPT
RSUB = NPAD // NS
CH = 64

_mesh = plsc.VectorSubcoreMesh(core_axis_name="c", subcore_axis_name="s")

_sc_params = pltpu.CompilerParams()
if "needs_layout_passes" in pltpu.CompilerParams.__dataclass_fields__:
    _sc_params = dataclasses.replace(_sc_params, needs_layout_passes=False)


def _mm_body(x_ref, w_ref, o_ref):
    o_ref[...] = jnp.dot(x_ref[...], w_ref[...],
                         preferred_element_type=jnp.float32,
                         precision=lax.Precision.HIGHEST)


def _mm(x, w):
    return pl.pallas_call(
        _mm_body,
        out_shape=jax.ShapeDtypeStruct((x.shape[0], w.shape[1]), jnp.float32),
    )(x, w)


def _res_mm_body(p_ref, w_ref, x1_ref, y2_ref):
    x1 = jnp.maximum(p_ref[0] + p_ref[1], 0.0)
    x1_ref[...] = x1
    y2_ref[...] = jnp.dot(x1, w_ref[...],
                          preferred_element_type=jnp.float32,
                          precision=lax.Precision.HIGHEST)


def _res_mm(p, w):
    return pl.pallas_call(
        _res_mm_body,
        out_shape=(jax.ShapeDtypeStruct((NPAD, D), jnp.float32),
                   jax.ShapeDtypeStruct((NPAD, D), jnp.float32)),
    )(p, w)


@functools.partial(
    pl.kernel,
    out_type=jax.ShapeDtypeStruct((NC, NPAD, D), jnp.float32),
    mesh=_mesh,
    scratch_types=[
        pltpu.VMEM((RPT, 128), jnp.int32),
        pltpu.VMEM((RPT, 128), jnp.int32),
        pltpu.VMEM((RPT, 128), jnp.float32),
        pltpu.VMEM((128, D), jnp.float32),
        pltpu.VMEM_SHARED((NPAD, D), jnp.float32),
        pltpu.SemaphoreType.DMA,
    ],
    compiler_params=_sc_params,
)
def _edge_pass(y_hbm, src_hbm, dst_hbm, cci_hbm, out_hbm,
               src_v, dst_v, cci_v, rows_v, acc, sem):
    c = lax.axis_index("c")
    s = lax.axis_index("s")
    w = s * NC + c

    zero = jnp.zeros((16,), jnp.float32)

    @pl.loop(0, 128)
    def _(r):
        for k in range(8):
            rows_v[r, pl.ds(16 * k, 16)] = zero

    base = s * RSUB
    for i in range(5):
        pltpu.sync_copy(rows_v, acc.at[pl.ds(base + i * 128, 128)])

    pltpu.sync_copy(src_hbm.at[w], src_v)
    pltpu.sync_copy(dst_hbm.at[w], dst_v)
    pltpu.sync_copy(cci_hbm.at[w], cci_v)

    plsc.subcore_barrier()

    @pl.loop(0, RPT)
    def _(j):
        pltpu.async_copy(y_hbm.at[src_v.at[j]], rows_v, sem).wait()

        jidx = jnp.full((16,), j, dtype=jnp.int32)

        @pl.loop(0, 128)
        def _(e):
            eidx = jnp.full((16,), e, dtype=jnp.int32)
            cvec = plsc.load_gather(cci_v, [jidx, eidx])
            for k in range(8):
                rows_v[e, pl.ds(16 * k, 16)] = (
                    rows_v[e, pl.ds(16 * k, 16)] * cvec)

        pltpu.sync_copy(rows_v, acc.at[dst_v.at[j]], add=True)

    plsc.subcore_barrier()

    for i in range(5):
        sl = pl.ds(base + i * 128, 128)
        pltpu.sync_copy(acc.at[sl], out_hbm.at[c].at[sl])


@functools.partial(
    pl.kernel,
    out_type=jax.ShapeDtypeStruct((NT, 8, D), jnp.float32),
    mesh=_mesh,
    scratch_types=[
        pltpu.VMEM((N,), jnp.int32),
        pltpu.VMEM((CH, D), jnp.float32),
        pltpu.VMEM((CH, D), jnp.float32),
        pltpu.VMEM((CH, D), jnp.float32),
        pltpu.VMEM((8, D), jnp.float32),
    ],
    compiler_params=_sc_params,
)
def _pool_pass(q_hbm, x1_hbm, batch_hbm, out_hbm,
               batch_v, q0_v, q1_v, x1_v, st_v):
    c = lax.axis_index("c")
    s = lax.axis_index("s")
    w = s * NC + c

    pltpu.sync_copy(batch_hbm, batch_v)

    def count_less(gval):
        def body(i, acc):
            v = batch_v[pl.ds(i * 16, 16)]
            return acc + jnp.sum(jnp.where(v < gval, 1, 0).astype(jnp.int32))
        return lax.fori_loop(0, N // 16, body, jnp.int32(0))

    g0 = 2 * w
    b0 = count_less(g0)
    b1 = count_less(g0 + 1)
    b2 = count_less(g0 + 2)

    neg_inf = jnp.full((16,), -jnp.inf, dtype=jnp.float32)
    pos_inf = jnp.full((16,), jnp.inf, dtype=jnp.float32)
    zero = jnp.zeros((16,), jnp.float32)

    for gi in range(2):
        lo = b0 if gi == 0 else b1
        hi = b1 if gi == 0 else b2

        for k in range(8):
            sl = pl.ds(16 * k, 16)
            st_v[4 * gi + 0, sl] = zero
            st_v[4 * gi + 1, sl] = zero
            st_v[4 * gi + 2, sl] = neg_inf
            st_v[4 * gi + 3, sl] = pos_inf

        abase = (lo // CH) * CH
        nch = (hi - abase + CH - 1) // CH

        def chunk_body(kk, _, lo=lo, hi=hi, abase=abase, gi=gi):
            start = abase + kk * CH
            startc = pl.multiple_of(jnp.minimum(start, N - CH), 8)
            pltpu.sync_copy(q_hbm.at[0].at[pl.ds(startc, CH)], q0_v)
            pltpu.sync_copy(q_hbm.at[1].at[pl.ds(startc, CH)], q1_v)
            pltpu.sync_copy(x1_hbm.at[pl.ds(startc, CH)], x1_v)
            lo_buf = jnp.maximum(lo, start) - startc
            hi_buf = jnp.minimum(start + CH, hi) - startc

            def row_body(r, _):
                for k in range(8):
                    sl = pl.ds(16 * k, 16)
                    v = jnp.maximum(q0_v[r, sl] + q1_v[r, sl] + x1_v[r, sl],
                                    0.0)
                    st_v[4 * gi + 0, sl] = st_v[4 * gi + 0, sl] + v
                    st_v[4 * gi + 1, sl] = st_v[4 * gi + 1, sl] + v * v
                    st_v[4 * gi + 2, sl] = jnp.maximum(st_v[4 * gi + 2, sl], v)
                    st_v[4 * gi + 3, sl] = jnp.minimum(st_v[4 * gi + 3, sl], v)
                return 0

            lax.fori_loop(lo_buf, hi_buf, row_body, 0)
            return 0

        lax.fori_loop(0, nch, chunk_body, 0)

    pltpu.sync_copy(st_v, out_hbm.at[w])


def _mlp_body(st_ref, batch_ref, gf_ref,
              w1_ref, b1_ref, w2_ref, b2_ref, w3_ref, b3_ref, w4_ref, b4_ref,
              o_ref):
    b = batch_ref[...]
    gids = lax.broadcasted_iota(jnp.int32, (1, G), 1)
    onehot = (b == gids).astype(jnp.float32)
    counts = jnp.sum(onehot, axis=0)[:, None]
    cnt = jnp.maximum(counts, 1.0)

    sum_ = st_ref[0]
    sq = st_ref[1]
    mx = st_ref[2]
    mn = st_ref[3]
    avg = sum_ / cnt
    var = jnp.maximum(sq / cnt - avg * avg, 0.0)
    std = jnp.sqrt(var + 1e-06)

    z = jnp.concatenate([avg, std, mx, mn, gf_ref[...]], axis=1)
    hp = dict(preferred_element_type=jnp.float32,
              precision=lax.Precision.HIGHEST)
    z = jnp.maximum(jnp.dot(z, w1_ref[...], **hp) + b1_ref[...], 0.0)
    z = jnp.maximum(jnp.dot(z, w2_ref[...], **hp) + b2_ref[...], 0.0)
    z = jnp.maximum(jnp.dot(z, w3_ref[...], **hp) + b3_ref[...], 0.0)
    z = jnp.dot(z, w4_ref[...], **hp) + b4_ref[...]
    half = z.shape[1] // 2
    o_ref[...] = jnp.concatenate([z[:, :half], jnp.square(z[:, half:])],
                                 axis=1)


def _mlp(stats, batch2d, gf, fw1, fb1, fw2, fb2, fw3, fb3, fw4, fb4):
    return pl.pallas_call(
        _mlp_body,
        out_shape=jax.ShapeDtypeStruct((G, 2), jnp.float32),
    )(stats, batch2d, gf,
      fw1, fb1.reshape(1, -1), fw2, fb2.reshape(1, -1),
      fw3, fb3.reshape(1, -1), fw4, fb4.reshape(1, -1))


def kernel(x_0, n0_to_0, cci_0_to_0, global_feature, batch_0,
           W1, W2, fc1_w, fc1_b, fc2_w, fc2_b, fc3_w, fc3_b, fc4_w, fc4_b):
    src = n0_to_0[0].astype(jnp.int32)
    dst = n0_to_0[1].astype(jnp.int32)
    cci = cci_0_to_0.astype(jnp.float32)
    pad = EPAD - E
    src_p = jnp.concatenate(
        [src, jnp.zeros((pad,), jnp.int32)]).reshape(NT, RPT, 128)
    dst_p = jnp.concatenate(
        [dst, jnp.zeros((pad,), jnp.int32)]).reshape(NT, RPT, 128)
    cci_p = jnp.concatenate(
        [cci, jnp.zeros((pad,), jnp.float32)]).reshape(NT, RPT, 128)
    batch32 = batch_0.astype(jnp.int32)
    x0p = jnp.zeros((NPAD, D), jnp.float32).at[:N].set(x_0)

    y1 = _mm(x0p, W1)
    p = _edge_pass(y1, src_p, dst_p, cci_p)
    x1, y2 = _res_mm(p, W2)
    q = _edge_pass(y2, src_p, dst_p, cci_p)
    stats = _pool_pass(q, x1, batch32)
    stats4 = stats.reshape(NT, 2, 4, D).transpose(2, 0, 1, 3).reshape(4, G, D)
    out = _mlp(stats4, batch32.reshape(N, 1), global_feature,
               fc1_w, fc1_b, fc2_w, fc2_b, fc3_w, fc3_b, fc4_w, fc4_b)
    return out
